# Initial kernel scaffold; baseline (speedup 1.0000x reference)
import jax
import jax.numpy as jnp
from jax.experimental import pallas as pl


def kernel(x, edge_index, edge_attr, batch, params):
    raise NotImplementedError("write your pallas kernel here")


# speedup vs baseline: 52.8647x; 52.8647x over previous
import functools

import jax
import jax.numpy as jnp
from jax import lax
from jax.experimental import pallas as pl
from jax.experimental.pallas import tpu as pltpu
from jax.experimental.pallas import tpu_sc as plsc

N = 10000
E = 320000
NP = 10240
EP = 327680
NE = 64
NOUT1 = 16
NOUT2 = 8
NGRAPH = 8
TE = 512
TN = 512
NW = 32
CH = 128
PAD_DST = N + 16


def _edge_msg_body(ea_ref, xj_ref, w11, w12, w13a, w13b, wf, rep8, g8, out_ref):
    a = ea_ref[...]
    t1 = jnp.maximum(jnp.dot(a, w11[...], preferred_element_type=jnp.float32), 0.0)
    t2 = jnp.tanh(jnp.dot(a, w12[...], preferred_element_type=jnp.float32))
    ea = jnp.maximum(
        jnp.dot(t1, w13a[...], preferred_element_type=jnp.float32)
        + jnp.dot(t2, w13b[...], preferred_element_type=jnp.float32), 0.0)
    xv = xj_ref[...]
    r8 = rep8[...]
    acc = jnp.zeros((TE, 128), jnp.float32)
    for g in range(8):
        bg = jnp.dot(xv, wf[:, g * 128:(g + 1) * 128],
                     preferred_element_type=jnp.float32)
        eag = jnp.dot(ea[:, g * 8:(g + 1) * 8], r8,
                      preferred_element_type=jnp.float32)
        acc = acc + eag * bg
    out_ref[...] = jnp.dot(acc, g8[...], preferred_element_type=jnp.float32)


def _edge_msg(ea_p, xj, w11, w12, w13a, w13b, wf, rep8, g8, D):
    grid = (EP // TE,)
    full = lambda i: (0, 0)
    return pl.pallas_call(
        _edge_msg_body,
        grid=grid,
        in_specs=[
            pl.BlockSpec((TE, NE), lambda i: (i, 0)),
            pl.BlockSpec((TE, D), lambda i: (i, 0)),
            pl.BlockSpec((NE, 2 * NE), full),
            pl.BlockSpec((NE, 2 * NE), full),
            pl.BlockSpec((2 * NE, NE), full),
            pl.BlockSpec((2 * NE, NE), full),
            pl.BlockSpec((D, 1024), full),
            pl.BlockSpec((8, 128), full),
            pl.BlockSpec((128, 16), full),
        ],
        out_specs=pl.BlockSpec((TE, 16), lambda i: (i, 0)),
        out_shape=jax.ShapeDtypeStruct((EP, 16), jnp.float32),
    )(ea_p, xj, w11, w12, w13a, w13b, wf, rep8, g8)


def _node_body(cp_ref, x_ref, cb, w11, b11, w12, b12, out_ref):
    conv = cp_ref[0] + cp_ref[1]
    c = jnp.maximum(conv + cb[...], 0.0)
    xv = x_ref[...]
    g1 = jnp.tanh(jnp.dot(xv, w11[...], preferred_element_type=jnp.float32) + b11[...])
    g2 = jnp.tanh(jnp.dot(xv, w12[...], preferred_element_type=jnp.float32) + b12[...])
    out_ref[...] = jnp.concatenate(
        [c, g1 * g2, jnp.zeros((TN, 8), jnp.float32)], axis=1)


def _node_finalize(conv_part, x_p, cb, w11, b11, w12, b12, D):
    grid = (NP // TN,)
    full = lambda i: (0, 0)
    return pl.pallas_call(
        _node_body,
        grid=grid,
        in_specs=[
            pl.BlockSpec((2, TN, 16), lambda i: (0, i, 0)),
            pl.BlockSpec((TN, D), lambda i: (i, 0)),
            pl.BlockSpec((1, 16), full),
            pl.BlockSpec((D, 8), full),
            pl.BlockSpec((1, 8), full),
            pl.BlockSpec((D, 8), full),
            pl.BlockSpec((1, 8), full),
        ],
        out_specs=pl.BlockSpec((TN, 32), lambda i: (i, 0)),
        out_shape=jax.ShapeDtypeStruct((NP, 32), jnp.float32),
    )(conv_part, x_p, cb, w11, b11, w12, b12)


def _pool_body(h_ref, b_ref, fc1w, fc1b, fc2w, fc2b, out_ref, acc, cnt):
    i = pl.program_id(0)

    @pl.when(i == 0)
    def _():
        acc[...] = jnp.zeros((NGRAPH, 32), jnp.float32)
        cnt[...] = jnp.zeros((NGRAPH, 1), jnp.float32)

    gids = lax.broadcasted_iota(jnp.int32, (1, NGRAPH), 1)
    oh = (b_ref[...] == gids).astype(jnp.float32)
    acc[...] += lax.dot_general(oh, h_ref[...], (((0,), (0,)), ((), ())),
                                preferred_element_type=jnp.float32)
    cnt[...] += lax.dot_general(oh, jnp.ones((TN, 1), jnp.float32),
                                (((0,), (0,)), ((), ())),
                                preferred_element_type=jnp.float32)

    @pl.when(i == NP // TN - 1)
    def _():
        pooled = acc[...][:, :24] / jnp.maximum(cnt[...], 1.0)
        a1 = jnp.maximum(
            jnp.dot(pooled, fc1w[...], preferred_element_type=jnp.float32)
            + fc1b[...], 0.0)
        out_ref[...] = (jnp.dot(a1, fc2w[...], preferred_element_type=jnp.float32)
                        + fc2b[...])


def _pool_head(h3, batch_p, fc1w, fc1b, fc2w, fc2b):
    grid = (NP // TN,)
    full = lambda i: (0, 0)
    return pl.pallas_call(
        _pool_body,
        grid=grid,
        in_specs=[
            pl.BlockSpec((TN, 32), lambda i: (i, 0)),
            pl.BlockSpec((TN, 1), lambda i: (i, 0)),
            pl.BlockSpec((24, 10), full),
            pl.BlockSpec((1, 10), full),
            pl.BlockSpec((10, 1), full),
            pl.BlockSpec((1, 1), full),
        ],
        out_specs=pl.BlockSpec((NGRAPH, 1), full),
        out_shape=jax.ShapeDtypeStruct((NGRAPH, 1), jnp.float32),
        scratch_shapes=[
            pltpu.VMEM((NGRAPH, 32), jnp.float32),
            pltpu.VMEM((NGRAPH, 1), jnp.float32),
        ],
    )(h3, batch_p, fc1w, fc1b, fc2w, fc2b)


def _sc_gather(table, idx, D):
    per_w = EP // NW
    iters = per_w // CH
    mesh = plsc.VectorSubcoreMesh(core_axis_name="c", subcore_axis_name="s")

    @functools.partial(
        pl.kernel, mesh=mesh,
        out_type=jax.ShapeDtypeStruct((EP, D), jnp.float32),
        compiler_params=pltpu.CompilerParams(use_tc_tiling_on_sc=False),
        scratch_types=[
            pltpu.VMEM((CH,), jnp.int32),
            pltpu.VMEM((CH, D), jnp.float32),
            pltpu.SemaphoreType.DMA,
        ],
    )
    def k(tab_hbm, idx_hbm, out_hbm, idx_v, rows_v, sem):
        wid = lax.axis_index("s") * 2 + lax.axis_index("c")
        base = wid * per_w

        def body(j, _):
            off = base + j * CH
            pltpu.sync_copy(idx_hbm.at[pl.ds(off, CH)], idx_v)
            pltpu.async_copy(tab_hbm.at[idx_v], rows_v, sem).wait()
            pltpu.sync_copy(rows_v, out_hbm.at[pl.ds(off, CH)])
            return 0

        lax.fori_loop(0, iters, body, 0)

    return k(table, idx)


def _sc_scatter(msg, dst):
    per_w = EP // NW
    iters = per_w // CH
    rows = NP // 16
    mesh = plsc.VectorSubcoreMesh(core_axis_name="c", subcore_axis_name="s")

    @functools.partial(
        pl.kernel, mesh=mesh,
        out_type=jax.ShapeDtypeStruct((2 * NP, 16), jnp.float32),
        compiler_params=pltpu.CompilerParams(use_tc_tiling_on_sc=False),
        scratch_types=[
            pltpu.VMEM((CH,), jnp.int32),
            pltpu.VMEM((CH, 16), jnp.float32),
            pltpu.VMEM((rows, 16), jnp.float32),
            pltpu.VMEM_SHARED((NP, 16), jnp.float32),
        ],
    )
    def k(msg_hbm, dst_hbm, z_hbm, out_hbm, idx_v, m_v, buf_v, acc_sh):
        cid = lax.axis_index("c")
        sid = lax.axis_index("s")
        wid = sid * 2 + cid

        pltpu.sync_copy(z_hbm, buf_v)
        pltpu.sync_copy(buf_v, acc_sh.at[pl.ds(sid * rows, rows)])
        plsc.subcore_barrier()

        base = wid * per_w

        def body(j, _):
            off = base + j * CH
            pltpu.sync_copy(dst_hbm.at[pl.ds(off, CH)], idx_v)
            pltpu.sync_copy(msg_hbm.at[pl.ds(off, CH)], m_v)
            pltpu.sync_copy(m_v, acc_sh.at[idx_v], add=True)
            return 0

        lax.fori_loop(0, iters, body, 0)
        plsc.subcore_barrier()
        pltpu.sync_copy(acc_sh.at[pl.ds(sid * rows, rows)], buf_v)
        pltpu.sync_copy(buf_v, out_hbm.at[pl.ds(cid * NP + sid * rows, rows)])

    zeros = jnp.zeros((rows, 16), jnp.float32)
    return k(msg, dst, zeros).reshape(2, NP, 16)


def _prep_layer(p, D):
    w = p["conv_w"]
    ninp = w.shape[1]
    if ninp < D:
        w = jnp.pad(w, ((0, 0), (0, D - ninp), (0, 0)))
        w11 = jnp.pad(p["fc11_w"], ((0, D - ninp), (0, 0)))
        w12 = jnp.pad(p["fc12_w"], ((0, D - ninp), (0, 0)))
    else:
        w11, w12 = p["fc11_w"], p["fc12_w"]
    wf = w.transpose(1, 0, 2).reshape(D, 1024)
    return {
        "w11e": p["fc1_1"], "w12e": p["fc1_2"],
        "w13a": p["fc1_3"][:2 * NE], "w13b": p["fc1_3"][2 * NE:],
        "wf": wf, "cb": p["conv_b"].reshape(1, 16),
        "w11": w11, "b11": p["fc11_b"].reshape(1, 8),
        "w12": w12, "b12": p["fc12_b"].reshape(1, 8),
    }


def kernel(x, edge_index, edge_attr, batch, params):
    f32 = jnp.float32
    src = edge_index[0]
    dst = edge_index[1]
    src_p = jnp.pad(src, (0, EP - E))
    dst_p = jnp.pad(dst, (0, EP - E), constant_values=PAD_DST)
    batch_p = jnp.pad(batch, (0, NP - N), constant_values=NGRAPH).reshape(NP, 1)
    ea_p = jnp.pad(edge_attr, ((0, EP - E), (0, 0)))
    x_p = jnp.pad(x, ((0, NP - N), (0, 0)))

    j = jnp.arange(128)
    rep8 = (j[None, :] // 16 == jnp.arange(8)[:, None]).astype(f32)
    g8 = (j[:, None] % 16 == jnp.arange(16)[None, :]).astype(f32)

    h = x_p
    D = 128
    for lname in ("l1", "l2", "l3"):
        lp = _prep_layer(params[lname], D)
        xj = _sc_gather(h, src_p, D)
        msg = _edge_msg(ea_p, xj, lp["w11e"], lp["w12e"], lp["w13a"], lp["w13b"],
                        lp["wf"], rep8, g8, D)
        conv_part = _sc_scatter(msg, dst_p)
        h = _node_finalize(conv_part, h, lp["cb"], lp["w11"], lp["b11"],
                           lp["w12"], lp["b12"], D)
        D = 32

    return _pool_head(h, batch_p,
                      params["fc1_w"], params["fc1_b"].reshape(1, 10),
                      params["fc2_w"], params["fc2_b"].reshape(1, 1))


# speedup vs baseline: 76.3234x; 1.4437x over previous
import functools

import jax
import jax.numpy as jnp
from jax import lax
from jax.experimental import pallas as pl
from jax.experimental.pallas import tpu as pltpu
from jax.experimental.pallas import tpu_sc as plsc

N = 10000
E = 320000
NP = 10240
EP = 327680
NE = 64
NOUT1 = 16
NOUT2 = 8
NGRAPH = 8
TE = 4096
TN = 512
NW = 32
CH = 128
PAD_DST = N + 16


def _edge_msg_body(ea_ref, xj_ref, w11, w12, w13a, w13b, wf, rep8, g8, out_ref):
    bf16 = jnp.bfloat16
    a = ea_ref[...]
    t1 = jnp.maximum(jnp.dot(a, w11[...], preferred_element_type=jnp.float32), 0.0)
    t2 = jnp.tanh(jnp.dot(a, w12[...], preferred_element_type=jnp.float32))
    ea = jnp.maximum(
        jnp.dot(t1.astype(bf16), w13a[...], preferred_element_type=jnp.float32)
        + jnp.dot(t2.astype(bf16), w13b[...], preferred_element_type=jnp.float32),
        0.0).astype(bf16)
    xv = xj_ref[...].astype(bf16)
    r8 = rep8[...]
    acc = jnp.zeros((TE, 128), jnp.float32)
    for g in range(8):
        bg = jnp.dot(xv, wf[:, g * 128:(g + 1) * 128],
                     preferred_element_type=jnp.float32)
        eag = jnp.dot(ea[:, g * 8:(g + 1) * 8], r8,
                      preferred_element_type=jnp.float32)
        acc = acc + eag * bg
    out_ref[...] = jnp.dot(acc, g8[...], preferred_element_type=jnp.float32)


def _edge_msg(ea_p, xj, w11, w12, w13a, w13b, wf, rep8, g8, D):
    grid = (EP // TE,)
    full = lambda i: (0, 0)
    return pl.pallas_call(
        _edge_msg_body,
        grid=grid,
        in_specs=[
            pl.BlockSpec((TE, NE), lambda i: (i, 0)),
            pl.BlockSpec((TE, D), lambda i: (i, 0)),
            pl.BlockSpec((NE, 2 * NE), full),
            pl.BlockSpec((NE, 2 * NE), full),
            pl.BlockSpec((2 * NE, NE), full),
            pl.BlockSpec((2 * NE, NE), full),
            pl.BlockSpec((D, 1024), full),
            pl.BlockSpec((8, 128), full),
            pl.BlockSpec((128, 16), full),
        ],
        out_specs=pl.BlockSpec((TE, 16), lambda i: (i, 0)),
        out_shape=jax.ShapeDtypeStruct((EP, 16), jnp.float32),
    )(ea_p, xj, w11, w12, w13a, w13b, wf, rep8, g8)


def _node_body(cp_ref, x_ref, cb, w11, b11, w12, b12, out_ref):
    conv = cp_ref[0] + cp_ref[1]
    c = jnp.maximum(conv + cb[...], 0.0)
    xv = x_ref[...]
    g1 = jnp.tanh(jnp.dot(xv, w11[...], preferred_element_type=jnp.float32) + b11[...])
    g2 = jnp.tanh(jnp.dot(xv, w12[...], preferred_element_type=jnp.float32) + b12[...])
    out_ref[...] = jnp.concatenate(
        [c, g1 * g2, jnp.zeros((TN, 8), jnp.float32)], axis=1)


def _node_finalize(conv_part, x_p, cb, w11, b11, w12, b12, D):
    grid = (NP // TN,)
    full = lambda i: (0, 0)
    return pl.pallas_call(
        _node_body,
        grid=grid,
        in_specs=[
            pl.BlockSpec((2, TN, 16), lambda i: (0, i, 0)),
            pl.BlockSpec((TN, D), lambda i: (i, 0)),
            pl.BlockSpec((1, 16), full),
            pl.BlockSpec((D, 8), full),
            pl.BlockSpec((1, 8), full),
            pl.BlockSpec((D, 8), full),
            pl.BlockSpec((1, 8), full),
        ],
        out_specs=pl.BlockSpec((TN, 32), lambda i: (i, 0)),
        out_shape=jax.ShapeDtypeStruct((NP, 32), jnp.float32),
    )(conv_part, x_p, cb, w11, b11, w12, b12)


def _pool_body(h_ref, b_ref, fc1w, fc1b, fc2w, fc2b, out_ref, acc, cnt):
    i = pl.program_id(0)

    @pl.when(i == 0)
    def _():
        acc[...] = jnp.zeros((NGRAPH, 32), jnp.float32)
        cnt[...] = jnp.zeros((NGRAPH, 1), jnp.float32)

    gids = lax.broadcasted_iota(jnp.int32, (1, NGRAPH), 1)
    oh = (b_ref[...] == gids).astype(jnp.float32)
    acc[...] += lax.dot_general(oh, h_ref[...], (((0,), (0,)), ((), ())),
                                preferred_element_type=jnp.float32)
    cnt[...] += lax.dot_general(oh, jnp.ones((TN, 1), jnp.float32),
                                (((0,), (0,)), ((), ())),
                                preferred_element_type=jnp.float32)

    @pl.when(i == NP // TN - 1)
    def _():
        pooled = acc[...][:, :24] / jnp.maximum(cnt[...], 1.0)
        a1 = jnp.maximum(
            jnp.dot(pooled, fc1w[...], preferred_element_type=jnp.float32)
            + fc1b[...], 0.0)
        out_ref[...] = (jnp.dot(a1, fc2w[...], preferred_element_type=jnp.float32)
                        + fc2b[...])


def _pool_head(h3, batch_p, fc1w, fc1b, fc2w, fc2b):
    grid = (NP // TN,)
    full = lambda i: (0, 0)
    return pl.pallas_call(
        _pool_body,
        grid=grid,
        in_specs=[
            pl.BlockSpec((TN, 32), lambda i: (i, 0)),
            pl.BlockSpec((TN, 1), lambda i: (i, 0)),
            pl.BlockSpec((24, 10), full),
            pl.BlockSpec((1, 10), full),
            pl.BlockSpec((10, 1), full),
            pl.BlockSpec((1, 1), full),
        ],
        out_specs=pl.BlockSpec((NGRAPH, 1), full),
        out_shape=jax.ShapeDtypeStruct((NGRAPH, 1), jnp.float32),
        scratch_shapes=[
            pltpu.VMEM((NGRAPH, 32), jnp.float32),
            pltpu.VMEM((NGRAPH, 1), jnp.float32),
        ],
    )(h3, batch_p, fc1w, fc1b, fc2w, fc2b)


def _sc_gather(table, idx2, D):
    chunks = EP // CH // NW
    K = 4 if D == 128 else 8
    G = chunks // K
    mesh = plsc.VectorSubcoreMesh(core_axis_name="c", subcore_axis_name="s")

    @functools.partial(
        pl.kernel, mesh=mesh,
        out_type=jax.ShapeDtypeStruct((EP, D), jnp.float32),
        compiler_params=pltpu.CompilerParams(use_tc_tiling_on_sc=False),
        scratch_types=[
            pltpu.VMEM((chunks, CH), jnp.int32),
            pltpu.VMEM((K, CH, D), jnp.float32),
            pltpu.SemaphoreType.DMA,
            pltpu.SemaphoreType.DMA,
        ],
    )
    def k(tab_hbm, idx_hbm, out_hbm, idx_v, bufs, gsem, wsem):
        wid = lax.axis_index("s") * 2 + lax.axis_index("c")
        crow = wid * chunks
        base = wid * chunks * CH
        pltpu.sync_copy(idx_hbm.at[pl.ds(crow, chunks)], idx_v)

        def grp(g, _):
            gets = [pltpu.async_copy(tab_hbm.at[idx_v.at[g * K + b]],
                                     bufs.at[b], gsem) for b in range(K)]
            for d in gets:
                d.wait()
            puts = [pltpu.async_copy(bufs.at[b],
                                     out_hbm.at[pl.ds(base + (g * K + b) * CH, CH)],
                                     wsem) for b in range(K)]
            for d in puts:
                d.wait()
            return 0

        lax.fori_loop(0, G, grp, 0)

    return k(table, idx2)


KS = 8


def _sc_scatter(msg, dst2):
    per_w = EP // NW
    rows = NP // 16
    mesh = plsc.VectorSubcoreMesh(core_axis_name="c", subcore_axis_name="s")

    @functools.partial(
        pl.kernel, mesh=mesh,
        out_type=jax.ShapeDtypeStruct((2 * NP, 16), jnp.float32),
        compiler_params=pltpu.CompilerParams(use_tc_tiling_on_sc=False),
        scratch_types=[
            pltpu.VMEM((per_w // CH, CH), jnp.int32),
            pltpu.VMEM((KS, CH, 16), jnp.float32),
            pltpu.VMEM((rows, 16), jnp.float32),
            pltpu.VMEM_SHARED((NP, 16), jnp.float32),
            pltpu.SemaphoreType.DMA,
            pltpu.SemaphoreType.DMA,
        ],
    )
    def k(msg_hbm, dst_hbm, z_hbm, out_hbm, idx_v, m_v, buf_v, acc_sh, msem, asem):
        cid = lax.axis_index("c")
        sid = lax.axis_index("s")
        wid = sid * 2 + cid
        crow = wid * (per_w // CH)
        base = wid * per_w

        pltpu.sync_copy(z_hbm, buf_v)
        pltpu.sync_copy(buf_v, acc_sh.at[pl.ds(sid * rows, rows)])
        pltpu.sync_copy(dst_hbm.at[pl.ds(crow, per_w // CH)], idx_v)
        plsc.subcore_barrier()

        def grp(g, _):
            gets = [pltpu.async_copy(
                msg_hbm.at[pl.ds(base + (g * KS + b) * CH, CH)],
                m_v.at[b], msem) for b in range(KS)]
            for d in gets:
                d.wait()
            adds = [pltpu.async_copy(m_v.at[b], acc_sh.at[idx_v.at[g * KS + b]],
                                     asem, add=True) for b in range(KS)]
            for d in adds:
                d.wait()
            return 0

        lax.fori_loop(0, (per_w // CH) // KS, grp, 0)
        plsc.subcore_barrier()
        pltpu.sync_copy(acc_sh.at[pl.ds(sid * rows, rows)], buf_v)
        pltpu.sync_copy(buf_v, out_hbm.at[pl.ds(cid * NP + sid * rows, rows)])

    zeros = jnp.zeros((rows, 16), jnp.float32)
    return k(msg, dst2, zeros).reshape(2, NP, 16)


def _prep_layer(p, D):
    w = p["conv_w"]
    ninp = w.shape[1]
    if ninp < D:
        w = jnp.pad(w, ((0, 0), (0, D - ninp), (0, 0)))
        w11 = jnp.pad(p["fc11_w"], ((0, D - ninp), (0, 0)))
        w12 = jnp.pad(p["fc12_w"], ((0, D - ninp), (0, 0)))
    else:
        w11, w12 = p["fc11_w"], p["fc12_w"]
    bf16 = jnp.bfloat16
    wf = w.transpose(1, 0, 2).reshape(D, 1024)
    return {
        "w11e": p["fc1_1"].astype(bf16), "w12e": p["fc1_2"].astype(bf16),
        "w13a": p["fc1_3"][:2 * NE].astype(bf16),
        "w13b": p["fc1_3"][2 * NE:].astype(bf16),
        "wf": wf.astype(bf16), "cb": p["conv_b"].reshape(1, 16),
        "w11": w11, "b11": p["fc11_b"].reshape(1, 8),
        "w12": w12, "b12": p["fc12_b"].reshape(1, 8),
    }


def kernel(x, edge_index, edge_attr, batch, params):
    f32 = jnp.float32
    bf16 = jnp.bfloat16
    src = edge_index[0]
    dst = edge_index[1]
    src2 = jnp.pad(src, (0, EP - E)).reshape(EP // CH, CH)
    dst2 = jnp.pad(dst, (0, EP - E), constant_values=PAD_DST).reshape(EP // CH, CH)
    batch_p = jnp.pad(batch, (0, NP - N), constant_values=NGRAPH).reshape(NP, 1)
    ea_p = jnp.pad(edge_attr.astype(bf16), ((0, EP - E), (0, 0)))
    x_p = jnp.pad(x, ((0, NP - N), (0, 0)))

    j = jnp.arange(128)
    rep8 = (j[None, :] // 16 == jnp.arange(8)[:, None]).astype(bf16)
    g8 = (j[:, None] % 16 == jnp.arange(16)[None, :]).astype(f32)

    h = x_p
    D = 128
    for lname in ("l1", "l2", "l3"):
        lp = _prep_layer(params[lname], D)
        xj = _sc_gather(h, src2, D)
        msg = _edge_msg(ea_p, xj, lp["w11e"], lp["w12e"], lp["w13a"], lp["w13b"],
                        lp["wf"], rep8, g8, D)
        conv_part = _sc_scatter(msg, dst2)
        h = _node_finalize(conv_part, h, lp["cb"], lp["w11"], lp["b11"],
                           lp["w12"], lp["b12"], D)
        D = 32

    return _pool_head(h, batch_p,
                      params["fc1_w"], params["fc1_b"].reshape(1, 10),
                      params["fc2_w"], params["fc2_b"].reshape(1, 1))


# speedup vs baseline: 85.1589x; 1.1158x over previous
import functools

import jax
import jax.numpy as jnp
from jax import lax
from jax.experimental import pallas as pl
from jax.experimental.pallas import tpu as pltpu
from jax.experimental.pallas import tpu_sc as plsc

N = 10000
E = 320000
NP = 10240
EP = 327680
NE = 64
NOUT1 = 16
NOUT2 = 8
NGRAPH = 8
TE = 4096
TN = 512
NW = 32
CH = 128
PAD_DST = N + 16


def _edge_msg_body(ea_ref, xj_ref, w11, w12, w13a, w13b, wf, rep8, g8, out_ref):
    bf16 = jnp.bfloat16
    a = ea_ref[...]
    t1 = jnp.maximum(jnp.dot(a, w11[...], preferred_element_type=jnp.float32), 0.0)
    t2 = jnp.tanh(jnp.dot(a, w12[...], preferred_element_type=jnp.float32))
    ea = jnp.maximum(
        jnp.dot(t1.astype(bf16), w13a[...], preferred_element_type=jnp.float32)
        + jnp.dot(t2.astype(bf16), w13b[...], preferred_element_type=jnp.float32),
        0.0).astype(bf16)
    xv = xj_ref[...]
    r8 = rep8[...]
    acc = jnp.zeros((TE, 128), jnp.float32)
    for g in range(8):
        bg = jnp.dot(xv, wf[:, g * 128:(g + 1) * 128],
                     preferred_element_type=jnp.float32)
        eag = jnp.dot(ea[:, g * 8:(g + 1) * 8], r8,
                      preferred_element_type=jnp.float32)
        acc = acc + eag * bg
    out_ref[...] = jnp.dot(acc, g8[...], preferred_element_type=jnp.float32)


def _edge_msg_body_p(ea_ref, xj_ref, w11, w12, w13a, w13b, wk2, r32, g8, out_ref):
    bf16 = jnp.bfloat16
    a = ea_ref[...]
    t1 = jnp.maximum(jnp.dot(a, w11[...], preferred_element_type=jnp.float32), 0.0)
    t2 = jnp.tanh(jnp.dot(a, w12[...], preferred_element_type=jnp.float32))
    ea = jnp.maximum(
        jnp.dot(t1.astype(bf16), w13a[...], preferred_element_type=jnp.float32)
        + jnp.dot(t2.astype(bf16), w13b[...], preferred_element_type=jnp.float32),
        0.0).astype(bf16)
    xv = xj_ref[...]
    p = jnp.dot(ea, wk2[...], preferred_element_type=jnp.float32)
    acc = jnp.zeros((TE, 128), jnp.float32)
    for g in range(3):
        xg = jnp.dot(xv, r32[:, g * 128:(g + 1) * 128],
                     preferred_element_type=jnp.float32)
        acc = acc + xg * p[:, g * 128:(g + 1) * 128]
    out_ref[...] = jnp.dot(acc, g8[...], preferred_element_type=jnp.float32)


def _edge_msg(ea_p, xj, w11, w12, w13a, w13b, wf, rep8, g8, D):
    grid = (EP // TE,)
    full = lambda i: (0, 0)
    body = _edge_msg_body if D == 128 else _edge_msg_body_p
    wf_spec = (pl.BlockSpec((D, 1024), full) if D == 128
               else pl.BlockSpec((NE, 384), full))
    rep_spec = (pl.BlockSpec((8, 128), full) if D == 128
                else pl.BlockSpec((32, 384), full))
    return pl.pallas_call(
        body,
        grid=grid,
        in_specs=[
            pl.BlockSpec((TE, NE), lambda i: (i, 0)),
            pl.BlockSpec((TE, D), lambda i: (i, 0)),
            pl.BlockSpec((NE, 2 * NE), full),
            pl.BlockSpec((NE, 2 * NE), full),
            pl.BlockSpec((2 * NE, NE), full),
            pl.BlockSpec((2 * NE, NE), full),
            wf_spec,
            rep_spec,
            pl.BlockSpec((128, 16), full),
        ],
        out_specs=pl.BlockSpec((TE, 16), lambda i: (i, 0)),
        out_shape=jax.ShapeDtypeStruct((EP, 16), jnp.float32),
    )(ea_p, xj, w11, w12, w13a, w13b, wf, rep8, g8)


def _node_body(cp_ref, x_ref, cb, w11, b11, w12, b12, out_ref):
    conv = cp_ref[0] + cp_ref[1]
    c = jnp.maximum(conv + cb[...], 0.0)
    xv = x_ref[...]
    g1 = jnp.tanh(jnp.dot(xv, w11[...], preferred_element_type=jnp.float32) + b11[...])
    g2 = jnp.tanh(jnp.dot(xv, w12[...], preferred_element_type=jnp.float32) + b12[...])
    out_ref[...] = jnp.concatenate(
        [c, g1 * g2, jnp.zeros((TN, 8), jnp.float32)], axis=1).astype(jnp.bfloat16)


def _node_finalize(conv_part, x_p, cb, w11, b11, w12, b12, D):
    grid = (NP // TN,)
    full = lambda i: (0, 0)
    return pl.pallas_call(
        _node_body,
        grid=grid,
        in_specs=[
            pl.BlockSpec((2, TN, 16), lambda i: (0, i, 0)),
            pl.BlockSpec((TN, D), lambda i: (i, 0)),
            pl.BlockSpec((1, 16), full),
            pl.BlockSpec((D, 8), full),
            pl.BlockSpec((1, 8), full),
            pl.BlockSpec((D, 8), full),
            pl.BlockSpec((1, 8), full),
        ],
        out_specs=pl.BlockSpec((TN, 32), lambda i: (i, 0)),
        out_shape=jax.ShapeDtypeStruct((NP, 32), jnp.bfloat16),
    )(conv_part, x_p, cb, w11, b11, w12, b12)


def _pool_body(h_ref, b_ref, fc1w, fc1b, fc2w, fc2b, out_ref, acc, cnt):
    i = pl.program_id(0)

    @pl.when(i == 0)
    def _():
        acc[...] = jnp.zeros((NGRAPH, 32), jnp.float32)
        cnt[...] = jnp.zeros((NGRAPH, 1), jnp.float32)

    gids = lax.broadcasted_iota(jnp.int32, (1, NGRAPH), 1)
    oh = (b_ref[...] == gids).astype(jnp.bfloat16)
    acc[...] += lax.dot_general(oh, h_ref[...], (((0,), (0,)), ((), ())),
                                preferred_element_type=jnp.float32)
    cnt[...] += lax.dot_general(oh, jnp.ones((TN, 1), jnp.bfloat16),
                                (((0,), (0,)), ((), ())),
                                preferred_element_type=jnp.float32)

    @pl.when(i == NP // TN - 1)
    def _():
        pooled = acc[...][:, :24] / jnp.maximum(cnt[...], 1.0)
        a1 = jnp.maximum(
            jnp.dot(pooled, fc1w[...], preferred_element_type=jnp.float32)
            + fc1b[...], 0.0)
        out_ref[...] = (jnp.dot(a1, fc2w[...], preferred_element_type=jnp.float32)
                        + fc2b[...])


def _pool_head(h3, batch_p, fc1w, fc1b, fc2w, fc2b):
    grid = (NP // TN,)
    full = lambda i: (0, 0)
    return pl.pallas_call(
        _pool_body,
        grid=grid,
        in_specs=[
            pl.BlockSpec((TN, 32), lambda i: (i, 0)),
            pl.BlockSpec((TN, 1), lambda i: (i, 0)),
            pl.BlockSpec((24, 10), full),
            pl.BlockSpec((1, 10), full),
            pl.BlockSpec((10, 1), full),
            pl.BlockSpec((1, 1), full),
        ],
        out_specs=pl.BlockSpec((NGRAPH, 1), full),
        out_shape=jax.ShapeDtypeStruct((NGRAPH, 1), jnp.float32),
        scratch_shapes=[
            pltpu.VMEM((NGRAPH, 32), jnp.float32),
            pltpu.VMEM((NGRAPH, 1), jnp.float32),
        ],
    )(h3, batch_p, fc1w, fc1b, fc2w, fc2b)


def _sc_gather(table, idx2, D):
    chunks = EP // CH // NW
    K = 8 if D == 128 else 16
    G = chunks // K
    mesh = plsc.VectorSubcoreMesh(core_axis_name="c", subcore_axis_name="s")

    @functools.partial(
        pl.kernel, mesh=mesh,
        out_type=jax.ShapeDtypeStruct((EP, D), jnp.bfloat16),
        compiler_params=pltpu.CompilerParams(use_tc_tiling_on_sc=False),
        scratch_types=[
            pltpu.VMEM((chunks, CH), jnp.int32),
            pltpu.VMEM((K, CH, D), jnp.bfloat16),
            pltpu.SemaphoreType.DMA,
            pltpu.SemaphoreType.DMA,
        ],
    )
    def k(tab_hbm, idx_hbm, out_hbm, idx_v, bufs, gsem, wsem):
        wid = lax.axis_index("s") * 2 + lax.axis_index("c")
        crow = wid * chunks
        base = wid * chunks * CH
        pltpu.sync_copy(idx_hbm.at[pl.ds(crow, chunks)], idx_v)

        def grp(g, _):
            gets = [pltpu.async_copy(tab_hbm.at[idx_v.at[g * K + b]],
                                     bufs.at[b], gsem) for b in range(K)]
            for d in gets:
                d.wait()
            puts = [pltpu.async_copy(bufs.at[b],
                                     out_hbm.at[pl.ds(base + (g * K + b) * CH, CH)],
                                     wsem) for b in range(K)]
            for d in puts:
                d.wait()
            return 0

        lax.fori_loop(0, G, grp, 0)

    return k(table, idx2)


KS = 8


def _sc_scatter(msg, dst2):
    per_w = EP // NW
    rows = NP // 16
    mesh = plsc.VectorSubcoreMesh(core_axis_name="c", subcore_axis_name="s")

    @functools.partial(
        pl.kernel, mesh=mesh,
        out_type=jax.ShapeDtypeStruct((2 * NP, 16), jnp.float32),
        compiler_params=pltpu.CompilerParams(use_tc_tiling_on_sc=False),
        scratch_types=[
            pltpu.VMEM((per_w // CH, CH), jnp.int32),
            pltpu.VMEM((KS, CH, 16), jnp.float32),
            pltpu.VMEM((rows, 16), jnp.float32),
            pltpu.VMEM_SHARED((NP, 16), jnp.float32),
            pltpu.SemaphoreType.DMA,
            pltpu.SemaphoreType.DMA,
        ],
    )
    def k(msg_hbm, dst_hbm, z_hbm, out_hbm, idx_v, m_v, buf_v, acc_sh, msem, asem):
        cid = lax.axis_index("c")
        sid = lax.axis_index("s")
        wid = sid * 2 + cid
        crow = wid * (per_w // CH)
        base = wid * per_w

        pltpu.sync_copy(z_hbm, buf_v)
        pltpu.sync_copy(buf_v, acc_sh.at[pl.ds(sid * rows, rows)])
        pltpu.sync_copy(dst_hbm.at[pl.ds(crow, per_w // CH)], idx_v)
        plsc.subcore_barrier()

        def grp(g, _):
            gets = [pltpu.async_copy(
                msg_hbm.at[pl.ds(base + (g * KS + b) * CH, CH)],
                m_v.at[b], msem) for b in range(KS)]
            for d in gets:
                d.wait()
            adds = [pltpu.async_copy(m_v.at[b], acc_sh.at[idx_v.at[g * KS + b]],
                                     asem, add=True) for b in range(KS)]
            for d in adds:
                d.wait()
            return 0

        lax.fori_loop(0, (per_w // CH) // KS, grp, 0)
        plsc.subcore_barrier()
        pltpu.sync_copy(acc_sh.at[pl.ds(sid * rows, rows)], buf_v)
        pltpu.sync_copy(buf_v, out_hbm.at[pl.ds(cid * NP + sid * rows, rows)])

    zeros = jnp.zeros((rows, 16), jnp.float32)
    return k(msg, dst2, zeros).reshape(2, NP, 16)


def _prep_layer(p, D):
    w = p["conv_w"]
    ninp = w.shape[1]
    if ninp < D:
        w = jnp.pad(w, ((0, 0), (0, D - ninp), (0, 0)))
        w11 = jnp.pad(p["fc11_w"], ((0, D - ninp), (0, 0)))
        w12 = jnp.pad(p["fc12_w"], ((0, D - ninp), (0, 0)))
    else:
        w11, w12 = p["fc11_w"], p["fc12_w"]
    bf16 = jnp.bfloat16
    if D == 128:
        wf = w.transpose(1, 0, 2).reshape(D, 1024)
    else:
        wf = p["conv_w"].reshape(NE, 24 * 16)
    return {
        "w11e": p["fc1_1"].astype(bf16), "w12e": p["fc1_2"].astype(bf16),
        "w13a": p["fc1_3"][:2 * NE].astype(bf16),
        "w13b": p["fc1_3"][2 * NE:].astype(bf16),
        "wf": wf.astype(bf16), "cb": p["conv_b"].reshape(1, 16),
        "w11": w11.astype(bf16), "b11": p["fc11_b"].reshape(1, 8),
        "w12": w12.astype(bf16), "b12": p["fc12_b"].reshape(1, 8),
    }


def kernel(x, edge_index, edge_attr, batch, params):
    f32 = jnp.float32
    bf16 = jnp.bfloat16
    src = edge_index[0]
    dst = edge_index[1]
    src2 = jnp.pad(src, (0, EP - E)).reshape(EP // CH, CH)
    dst2 = jnp.pad(dst, (0, EP - E), constant_values=PAD_DST).reshape(EP // CH, CH)
    batch_p = jnp.pad(batch, (0, NP - N), constant_values=NGRAPH).reshape(NP, 1)
    ea_p = jnp.pad(edge_attr.astype(bf16), ((0, EP - E), (0, 0)))
    x_p = jnp.pad(x.astype(bf16), ((0, NP - N), (0, 0)))

    j = jnp.arange(128)
    rep8 = (j[None, :] // 16 == jnp.arange(8)[:, None]).astype(bf16)
    c = jnp.arange(384)
    r32 = (jnp.arange(32)[:, None] ==
           8 * (c // 128) + (c % 128) // 16).astype(bf16)
    g8 = (j[:, None] % 16 == jnp.arange(16)[None, :]).astype(f32)

    h = x_p
    D = 128
    for lname in ("l1", "l2", "l3"):
        lp = _prep_layer(params[lname], D)
        xj = _sc_gather(h, src2, D)
        rep = rep8 if D == 128 else r32
        msg = _edge_msg(ea_p, xj, lp["w11e"], lp["w12e"], lp["w13a"], lp["w13b"],
                        lp["wf"], rep, g8, D)
        conv_part = _sc_scatter(msg, dst2)
        h = _node_finalize(conv_part, h, lp["cb"], lp["w11"], lp["b11"],
                           lp["w12"], lp["b12"], D)
        D = 32

    return _pool_head(h, batch_p,
                      params["fc1_w"], params["fc1_b"].reshape(1, 10),
                      params["fc2_w"], params["fc2_b"].reshape(1, 1))
